# baseline (speedup 1.0000x reference)
import jax
import jax.numpy as jnp
from jax import lax
from jax.experimental import pallas as pl
from jax.experimental.pallas import tpu as pltpu
from jax.experimental.pallas import tpu_sc as plsc

_NC, _NS = 2, 16
_NW = _NC * _NS
_B, _S, _D = 4096, 200, 64
_TABLE_ROWS = 101
_TLEN = _TABLE_ROWS * _D
_MROWS = 256
_DC = 4
_CLEN = _DC * _B
_NCH = (_S * _D) // _DC // _NW
_NB = 4
_NG = _B // 16


def _body(x_hbm, m_hbm, t_hbm, out_hbm, tabv, vlbuf, tmpv,
          rbuf0, rbuf1, rbuf2, rbuf3, shared, in_sem, out_sem):
    rbufs = [rbuf0, rbuf1, rbuf2, rbuf3]
    cid = lax.axis_index("c")
    sid = lax.axis_index("s")
    wid = sid * _NC + cid

    pltpu.sync_copy(t_hbm, tabv)
    zf = jnp.zeros((16,), jnp.float32)
    for g in range(4):
        tabv[pl.ds(g * 16, 16)] = zf

    for sl in range(2):
        rt = sid * 2 + sl
        pltpu.sync_copy(m_hbm.at[pl.ds(rt * 32768, _CLEN)], rbuf0)
        pltpu.sync_copy(m_hbm.at[pl.ds(rt * 32768 + _CLEN, _CLEN)], rbuf1)

        def _pg(g, carry, sl=sl):
            off = (g // 8) * 1024 + (g % 8) * 16
            acc0 = rbuf0[pl.ds(off, 16)]
            acc1 = rbuf1[pl.ds(off, 16)]
            for rr in range(1, 8):
                acc0 = acc0 + rbuf0[pl.ds(off + rr * 128, 16)]
                acc1 = acc1 + rbuf1[pl.ds(off + rr * 128, 16)]
            if sl == 0:
                vlbuf[pl.ds(g * 16, 16)] = acc0
                vlbuf[pl.ds((g + 128) * 16, 16)] = acc1
            else:
                vlbuf[pl.ds(g * 16, 16)] = vlbuf[pl.ds(g * 16, 16)] + acc0
                vlbuf[pl.ds((g + 128) * 16, 16)] = vlbuf[pl.ds((g + 128) * 16, 16)] + acc1
            return carry

        lax.fori_loop(0, 128, _pg, 0)

    pltpu.sync_copy(vlbuf, shared.at[sid])
    plsc.subcore_barrier()
    for k in range(_NS):
        pltpu.sync_copy(shared.at[k], tmpv)

        def _acc(g, carry, first=(k == 0)):
            if first:
                vlbuf[pl.ds(g * 16, 16)] = tmpv[pl.ds(g * 16, 16)]
            else:
                vlbuf[pl.ds(g * 16, 16)] = vlbuf[pl.ds(g * 16, 16)] + tmpv[pl.ds(g * 16, 16)]
            return carry

        lax.fori_loop(0, _NG, _acc, 0)

    cid0 = wid * _NCH

    def _start_in(c, b):
        pltpu.async_copy(x_hbm.at[pl.ds((cid0 + c) * _CLEN, _CLEN)], rbufs[b],
                         in_sem.at[b])

    def _wait_out(b):
        pltpu.make_async_copy(rbufs[b], out_hbm.at[pl.ds(0, _CLEN)],
                              out_sem.at[b]).wait()

    def _chunk(c, b):
        pltpu.make_async_copy(x_hbm.at[pl.ds(0, _CLEN)], rbufs[b],
                              in_sem.at[b]).wait()
        chunk = cid0 + c
        slab = chunk // 2
        half = chunk % 2
        p = slab // 8
        dt = slab % 8
        rb = rbufs[b]

        def _grp(bt, carry):
            for br0 in range(8):
                g = half * 128 + bt * 8 + br0
                vl = vlbuf[pl.ds(g * 16, 16)].astype(jnp.int32)
                s = jnp.maximum(jnp.minimum(vl - p, p + 1), 0)
                sidx = s * _D + dt * 8
                for dr in range(8):
                    off = bt * 1024 + dr * 128 + br0 * 16
                    pe = plsc.load_gather(tabv, [sidx + dr])
                    rb[pl.ds(off, 16)] = rb[pl.ds(off, 16)] + pe
            return carry

        lax.fori_loop(0, 16, _grp, 0)
        pltpu.async_copy(rb, out_hbm.at[pl.ds(chunk * _CLEN, _CLEN)],
                         out_sem.at[b])

    _start_in(0, 0)
    _start_in(1, 1)

    @pl.loop(0, _NCH, step=_NB)
    def _outer(g):
        for b in range(_NB):
            c = g + b
            nc = c + 2
            bb = (b + 2) % _NB
            if b < 2:
                @pl.when(g > 0)
                def _w():
                    _wait_out(bb)
                    _start_in(nc, bb)

                @pl.when(g == 0)
                def _s():
                    _start_in(nc, bb)
            else:
                @pl.when(g < _NCH - _NB)
                def _ws():
                    _wait_out(bb)
                    _start_in(nc, bb)
            _chunk(c, b)

    for b in range(_NB):
        _wait_out(b)


def kernel(x, mask, position_embedding):
    b, s, d = x.shape
    xf = (x.transpose(1, 2, 0).reshape(s, 8, d // 8, b // 128, 128)
          .transpose(0, 1, 3, 2, 4).reshape(-1))
    mt = (jnp.pad(mask.astype(jnp.float32).T, ((0, _MROWS - s), (0, 0)))
          .reshape(_MROWS // 8, 8, b // 128, 128).transpose(0, 2, 1, 3).reshape(-1))
    tf = position_embedding.reshape(-1)
    mesh = plsc.VectorSubcoreMesh(
        core_axis_name="c", subcore_axis_name="s", num_cores=_NC, num_subcores=_NS
    )
    outf = pl.kernel(
        _body,
        out_type=jax.ShapeDtypeStruct((s * d * b,), jnp.float32),
        mesh=mesh,
        compiler_params=pltpu.CompilerParams(needs_layout_passes=False),
        scratch_types=[
            pltpu.VMEM((_TLEN,), jnp.float32),
            pltpu.VMEM((_NG * 16,), jnp.float32),
            pltpu.VMEM((_NG * 16,), jnp.float32),
            pltpu.VMEM((_CLEN,), jnp.float32),
            pltpu.VMEM((_CLEN,), jnp.float32),
            pltpu.VMEM((_CLEN,), jnp.float32),
            pltpu.VMEM((_CLEN,), jnp.float32),
            pltpu.VMEM_SHARED((_NS, _NG * 16), jnp.float32),
            pltpu.SemaphoreType.DMA((_NB,)),
            pltpu.SemaphoreType.DMA((_NB,)),
        ],
    )(xf, mt, tf)
    out = (outf.reshape(s, 8, b // 128, d // 8, 128).transpose(0, 1, 3, 2, 4)
           .reshape(s, d, b).transpose(2, 0, 1))
    return out


# speedup vs baseline: 1.8554x; 1.8554x over previous
import jax
import jax.numpy as jnp
from jax import lax
from jax.experimental import pallas as pl
from jax.experimental.pallas import tpu as pltpu
from jax.experimental.pallas import tpu_sc as plsc

_NC, _NS = 2, 16
_NW = _NC * _NS
_B, _S, _D = 4096, 200, 64
_TABLE_ROWS = 101
_TLEN = _TABLE_ROWS * _D
_MROWS = 256
_DC = 4
_CLEN = _DC * _B
_NCH = (_S * _D) // _DC // _NW
_NB = 4
_NG = _B // 16


def _body(x_hbm, m_hbm, t_hbm, out_hbm, tabv, vlbuf, tmpv,
          rbuf0, rbuf1, rbuf2, rbuf3, shared, in_sem, out_sem):
    rbufs = [rbuf0, rbuf1, rbuf2, rbuf3]
    cid = lax.axis_index("c")
    sid = lax.axis_index("s")
    wid = sid * _NC + cid

    pltpu.sync_copy(t_hbm, tabv)
    zf = jnp.zeros((16,), jnp.float32)
    for g in range(4):
        tabv[pl.ds(g * 16, 16)] = zf

    for sl in range(2):
        rt = sid * 2 + sl
        pltpu.sync_copy(m_hbm.at[pl.ds(rt * 32768, _CLEN)], rbuf0)
        pltpu.sync_copy(m_hbm.at[pl.ds(rt * 32768 + _CLEN, _CLEN)], rbuf1)

        def _pg(g, carry, sl=sl):
            off = (g // 8) * 1024 + (g % 8) * 16
            acc0 = rbuf0[pl.ds(off, 16)]
            acc1 = rbuf1[pl.ds(off, 16)]
            for rr in range(1, 8):
                acc0 = acc0 + rbuf0[pl.ds(off + rr * 128, 16)]
                acc1 = acc1 + rbuf1[pl.ds(off + rr * 128, 16)]
            if sl == 0:
                vlbuf[pl.ds(g * 16, 16)] = acc0
                vlbuf[pl.ds((g + 128) * 16, 16)] = acc1
            else:
                vlbuf[pl.ds(g * 16, 16)] = vlbuf[pl.ds(g * 16, 16)] + acc0
                vlbuf[pl.ds((g + 128) * 16, 16)] = vlbuf[pl.ds((g + 128) * 16, 16)] + acc1
            return carry

        lax.fori_loop(0, 128, _pg, 0)

    pltpu.sync_copy(vlbuf, shared.at[sid])
    plsc.subcore_barrier()
    for k in range(_NS):
        pltpu.sync_copy(shared.at[k], tmpv)

        def _acc(g, carry, first=(k == 0)):
            if first:
                vlbuf[pl.ds(g * 16, 16)] = tmpv[pl.ds(g * 16, 16)]
            else:
                vlbuf[pl.ds(g * 16, 16)] = vlbuf[pl.ds(g * 16, 16)] + tmpv[pl.ds(g * 16, 16)]
            return carry

        lax.fori_loop(0, _NG, _acc, 0)

    cid0 = wid * _NCH

    def _start_in(c, b):
        pltpu.async_copy(x_hbm.at[pl.ds((cid0 + c) * _CLEN, _CLEN)], rbufs[b],
                         in_sem.at[b])

    def _wait_out(b):
        pltpu.make_async_copy(rbufs[b], out_hbm.at[pl.ds(0, _CLEN)],
                              out_sem.at[b]).wait()

    def _chunk(c, b):
        pltpu.make_async_copy(x_hbm.at[pl.ds(0, _CLEN)], rbufs[b],
                              in_sem.at[b]).wait()
        chunk = cid0 + c
        slab = chunk // 2
        half = chunk % 2
        p = slab // 8
        dt = slab % 8
        rb = rbufs[b]

        @plsc.parallel_loop(0, 16, unroll=2)
        def _grp(bt):
            for br0 in range(8):
                g = half * 128 + bt * 8 + br0
                vl = vlbuf[pl.ds(g * 16, 16)].astype(jnp.int32)
                s = jnp.maximum(jnp.minimum(vl - p, p + 1), 0)
                sidx = s * _D + dt * 8
                for dr in range(8):
                    off = bt * 1024 + dr * 128 + br0 * 16
                    pe = plsc.load_gather(tabv, [sidx + dr])
                    plsc.addupdate(rb.at[pl.ds(off, 16)], pe)
        pltpu.async_copy(rb, out_hbm.at[pl.ds(chunk * _CLEN, _CLEN)],
                         out_sem.at[b])

    _start_in(0, 0)
    _start_in(1, 1)

    @pl.loop(0, _NCH, step=_NB)
    def _outer(g):
        for b in range(_NB):
            c = g + b
            nc = c + 2
            bb = (b + 2) % _NB
            if b < 2:
                @pl.when(g > 0)
                def _w():
                    _wait_out(bb)
                    _start_in(nc, bb)

                @pl.when(g == 0)
                def _s():
                    _start_in(nc, bb)
            else:
                @pl.when(g < _NCH - _NB)
                def _ws():
                    _wait_out(bb)
                    _start_in(nc, bb)
            _chunk(c, b)

    for b in range(_NB):
        _wait_out(b)


def kernel(x, mask, position_embedding):
    b, s, d = x.shape
    xf = (x.transpose(1, 2, 0).reshape(s, 8, d // 8, b // 128, 128)
          .transpose(0, 1, 3, 2, 4).reshape(-1))
    mt = (jnp.pad(mask.astype(jnp.float32).T, ((0, _MROWS - s), (0, 0)))
          .reshape(_MROWS // 8, 8, b // 128, 128).transpose(0, 2, 1, 3).reshape(-1))
    tf = position_embedding.reshape(-1)
    mesh = plsc.VectorSubcoreMesh(
        core_axis_name="c", subcore_axis_name="s", num_cores=_NC, num_subcores=_NS
    )
    outf = pl.kernel(
        _body,
        out_type=jax.ShapeDtypeStruct((s * d * b,), jnp.float32),
        mesh=mesh,
        compiler_params=pltpu.CompilerParams(needs_layout_passes=False),
        scratch_types=[
            pltpu.VMEM((_TLEN,), jnp.float32),
            pltpu.VMEM((_NG * 16,), jnp.float32),
            pltpu.VMEM((_NG * 16,), jnp.float32),
            pltpu.VMEM((_CLEN,), jnp.float32),
            pltpu.VMEM((_CLEN,), jnp.float32),
            pltpu.VMEM((_CLEN,), jnp.float32),
            pltpu.VMEM((_CLEN,), jnp.float32),
            pltpu.VMEM_SHARED((_NS, _NG * 16), jnp.float32),
            pltpu.SemaphoreType.DMA((_NB,)),
            pltpu.SemaphoreType.DMA((_NB,)),
        ],
    )(xf, mt, tf)
    out = (outf.reshape(s, 8, b // 128, d // 8, 128).transpose(0, 1, 3, 2, 4)
           .reshape(s, d, b).transpose(2, 0, 1))
    return out
